# baseline (speedup 1.0000x reference)
import jax
import jax.numpy as jnp
from jax.experimental import pallas as pl
from jax.experimental.pallas import tpu as pltpu

_B_BLK = 4


def _fused_kernel(q_ref, k_ref, v_ref, pt_ref, bias_ref,
                  w1_ref, b1_ref, w2_ref, b2_ref, w_out_ref, b_out_ref,
                  out_ref, weights_ref):
    H = q_ref.shape[1]
    L = q_ref.shape[2]
    Dk = q_ref.shape[3]
    scale = 1.0 / (Dk ** 0.5)
    w_out = w_out_ref[...]
    b_out = b_out_ref[...]

    pt = pt_ref[...].reshape(_B_BLK * H * L, Dk)
    hpt = jnp.tanh(jnp.dot(pt, w1_ref[...],
                           preferred_element_type=jnp.float32) + b1_ref[...])
    s = jnp.sum(hpt.reshape(_B_BLK * H, L, Dk) * w2_ref[0], axis=-1)
    gate_all = jax.nn.sigmoid(s + b2_ref[0, 0])

    for b in range(_B_BLK):
        q = q_ref[b]
        k = k_ref[b]
        v = v_ref[b]
        bias = bias_ref[b]
        gate = gate_all[b * H:(b + 1) * H]

        attn = jnp.einsum('hqd,hkd->hqk', q * scale, k,
                          preferred_element_type=jnp.float32) + bias

        m = jnp.max(attn, axis=-1, keepdims=True)
        e = jnp.exp(attn - m)
        denom = jnp.sum(e, axis=-1, keepdims=True)
        w = e * pl.reciprocal(denom, approx=True)
        w = w * gate[:, None, :]
        weights_ref[b] = w

        ctx = jnp.einsum('hqk,hkd->hqd', w, v,
                         preferred_element_type=jnp.float32)
        ctx_cat = jnp.concatenate([ctx[h] for h in range(H)], axis=-1)
        out_ref[b] = jnp.dot(ctx_cat, w_out,
                             preferred_element_type=jnp.float32) + b_out


def kernel(q, k, v, pt_attn, bias, w1, b1, w2, b2, w_out, b_out):
    B, H, Lq, Dk = q.shape
    Lk = k.shape[2]
    Dv = v.shape[3]
    d_model = H * Dv

    nblk = B // _B_BLK
    graph_out, weights = pl.pallas_call(
        _fused_kernel,
        out_shape=(jax.ShapeDtypeStruct((B, Lq, d_model), jnp.float32),
                   jax.ShapeDtypeStruct((B, H, Lq, Lk), jnp.float32)),
        grid=(nblk,),
        in_specs=[
            pl.BlockSpec((_B_BLK, H, Lq, Dk), lambda b: (b, 0, 0, 0)),
            pl.BlockSpec((_B_BLK, H, Lk, Dk), lambda b: (b, 0, 0, 0)),
            pl.BlockSpec((_B_BLK, H, Lk, Dv), lambda b: (b, 0, 0, 0)),
            pl.BlockSpec((_B_BLK, H, Lk, Dk), lambda b: (b, 0, 0, 0)),
            pl.BlockSpec((_B_BLK, H, Lq, Lk), lambda b: (b, 0, 0, 0)),
            pl.BlockSpec((Dk, Dv), lambda b: (0, 0)),
            pl.BlockSpec((1, Dv), lambda b: (0, 0)),
            pl.BlockSpec((1, Dv), lambda b: (0, 0)),
            pl.BlockSpec((1, 1), lambda b: (0, 0)),
            pl.BlockSpec((d_model, d_model), lambda b: (0, 0)),
            pl.BlockSpec((1, d_model), lambda b: (0, 0)),
        ],
        out_specs=(pl.BlockSpec((_B_BLK, Lq, d_model), lambda b: (b, 0, 0)),
                   pl.BlockSpec((_B_BLK, H, Lq, Lk), lambda b: (b, 0, 0, 0))),
        compiler_params=pltpu.CompilerParams(
            dimension_semantics=("parallel",),
            vmem_limit_bytes=100 * 1024 * 1024,
        ),
    )(q, k, v, pt_attn, bias, w1, b1, w2, b2, w_out, b_out)

    return graph_out, weights


# speedup vs baseline: 2.4685x; 2.4685x over previous
import jax
import jax.numpy as jnp
from jax.experimental import pallas as pl
from jax.experimental.pallas import tpu as pltpu

_B_BLK = 4


def _fused_kernel(qT_ref, kT_ref, vT_ref, ptT_ref, bias_ref,
                  w1_ref, b1c_ref, w2c_ref, b2_ref, w_out_ref, b_out_ref,
                  out_ref, weights_ref):
    H = qT_ref.shape[1]
    Dk = qT_ref.shape[2]
    scale = 1.0 / (Dk ** 0.5)
    w_out = w_out_ref[...]
    b_out = b_out_ref[...]
    w1 = w1_ref[...]
    b1c = b1c_ref[...]
    w2c = w2c_ref[...]
    b2 = b2_ref[0, 0]

    for b in range(_B_BLK):
        qT = qT_ref[b]
        kT = kT_ref[b]
        vT = vT_ref[b]
        ptT = ptT_ref[b]
        bias = bias_ref[b]

        gs = []
        for h in range(H):
            hptT = jnp.tanh(jax.lax.dot_general(
                w1, ptT[h], (((0,), (0,)), ((), ())),
                preferred_element_type=jnp.float32) + b1c)
            gs.append(jnp.sum(hptT * w2c, axis=0, keepdims=True))
        gate = jax.nn.sigmoid(jnp.concatenate(gs, axis=0) + b2)

        attn = jnp.einsum('hdq,hdk->hqk', qT * scale, kT,
                          preferred_element_type=jnp.float32) + bias
        m = jnp.max(attn, axis=-1, keepdims=True)
        e = jnp.exp(attn - m)
        denom = jnp.sum(e, axis=-1, keepdims=True)
        w = e * pl.reciprocal(denom, approx=True)
        w = w * gate[:, None, :]
        weights_ref[b] = w

        ctx_cat = jnp.concatenate(
            [jax.lax.dot_general(w[h], vT[h], (((1,), (1,)), ((), ())),
                                 preferred_element_type=jnp.float32)
             for h in range(H)], axis=-1)
        out_ref[b] = jnp.dot(ctx_cat, w_out,
                             preferred_element_type=jnp.float32) + b_out


def kernel(q, k, v, pt_attn, bias, w1, b1, w2, b2, w_out, b_out):
    B, H, Lq, Dk = q.shape
    Lk = k.shape[2]
    Dv = v.shape[3]
    d_model = H * Dv

    qT = jnp.swapaxes(q, 2, 3)
    kT = jnp.swapaxes(k, 2, 3)
    vT = jnp.swapaxes(v, 2, 3)
    ptT = jnp.swapaxes(pt_attn, 2, 3)
    b1c = b1.T
    w2c = w2.T

    nblk = B // _B_BLK
    graph_out, weights = pl.pallas_call(
        _fused_kernel,
        out_shape=(jax.ShapeDtypeStruct((B, Lq, d_model), jnp.float32),
                   jax.ShapeDtypeStruct((B, H, Lq, Lk), jnp.float32)),
        grid=(nblk,),
        in_specs=[
            pl.BlockSpec((_B_BLK, H, Dk, Lq), lambda b: (b, 0, 0, 0)),
            pl.BlockSpec((_B_BLK, H, Dk, Lk), lambda b: (b, 0, 0, 0)),
            pl.BlockSpec((_B_BLK, H, Dv, Lk), lambda b: (b, 0, 0, 0)),
            pl.BlockSpec((_B_BLK, H, Dk, Lk), lambda b: (b, 0, 0, 0)),
            pl.BlockSpec((_B_BLK, H, Lq, Lk), lambda b: (b, 0, 0, 0)),
            pl.BlockSpec((Dk, Dv), lambda b: (0, 0)),
            pl.BlockSpec((Dv, 1), lambda b: (0, 0)),
            pl.BlockSpec((Dv, 1), lambda b: (0, 0)),
            pl.BlockSpec((1, 1), lambda b: (0, 0)),
            pl.BlockSpec((d_model, d_model), lambda b: (0, 0)),
            pl.BlockSpec((1, d_model), lambda b: (0, 0)),
        ],
        out_specs=(pl.BlockSpec((_B_BLK, Lq, d_model), lambda b: (b, 0, 0)),
                   pl.BlockSpec((_B_BLK, H, Lq, Lk), lambda b: (b, 0, 0, 0))),
        compiler_params=pltpu.CompilerParams(
            dimension_semantics=("parallel",),
            vmem_limit_bytes=100 * 1024 * 1024,
        ),
    )(qT, kT, vT, ptT, bias, w1, b1c, w2c, b2, w_out, b_out)

    return graph_out, weights
